# Initial kernel scaffold; baseline (speedup 1.0000x reference)
import functools

import jax
import jax.numpy as jnp
from jax import lax
from jax.experimental import pallas as pl
from jax.experimental.pallas import tpu as pltpu
from jax.experimental.pallas import tpu_sc as plsc

NW = 32
CH = 128
NBUF = 8


def _gather_kernel(n_chunks, per_w, D, idx_hbm, table_hbm, out_hbm,
                   idx_v, rows_v, gsems, osems):
    nc = 2
    wid = lax.axis_index("s") * nc + lax.axis_index("c")
    pltpu.sync_copy(idx_hbm.at[wid], idx_v)
    base = wid * per_w

    @pl.loop(0, n_chunks, step=NBUF)
    def group(i0):
        descs = []
        for b in range(NBUF):
            @pl.when(i0 > 0)
            def _(b=b):
                pltpu.make_async_copy(
                    rows_v.at[b], out_hbm.at[pl.ds(0, CH)], osems[b]
                ).wait()
            descs.append(
                pltpu.async_copy(
                    table_hbm.at[idx_v.at[i0 + b]], rows_v.at[b], gsems[b]
                )
            )
        for b in range(NBUF):
            descs[b].wait()
            row0 = base + (i0 + b) * CH
            pltpu.async_copy(
                rows_v.at[b], out_hbm.at[pl.ds(row0, CH)], osems[b]
            )

    for b in range(NBUF):
        pltpu.make_async_copy(
            rows_v.at[b], out_hbm.at[pl.ds(0, CH)], osems[b]
        ).wait()


def kernel(x, table):
    B, H = x.shape
    V, D = table.shape
    N = B * H
    per_w = N // NW
    n_chunks = per_w // CH
    assert per_w * NW == N and n_chunks * CH == per_w and n_chunks % NBUF == 0

    idx = x.reshape(NW, n_chunks, CH).astype(jnp.int32)
    mesh = plsc.VectorSubcoreMesh(core_axis_name="c", subcore_axis_name="s")

    run = pl.kernel(
        functools.partial(_gather_kernel, n_chunks, per_w, D),
        out_type=jax.ShapeDtypeStruct((N, D), jnp.float32),
        mesh=mesh,
        scratch_types=[
            pltpu.VMEM((n_chunks, CH), jnp.int32),
            pltpu.VMEM((NBUF, CH, D), jnp.float32),
            [pltpu.SemaphoreType.DMA] * NBUF,
            [pltpu.SemaphoreType.DMA] * NBUF,
        ],
    )
    out = run(idx, table)
    return out.reshape(B, H, D)


# speedup vs baseline: 1.8725x; 1.8725x over previous
import functools

import jax
import jax.numpy as jnp
from jax import lax
from jax.experimental import pallas as pl
from jax.experimental.pallas import tpu as pltpu
from jax.experimental.pallas import tpu_sc as plsc

NW = 32
CH = 128
NBUF = 8


def _gather_kernel(n_chunks, per_w, D, idx_hbm, table_hbm, out_hbm,
                   idx_v, rows_v, gsems, osems):
    nc = 2
    wid = lax.axis_index("s") * nc + lax.axis_index("c")
    pltpu.sync_copy(idx_hbm.at[wid], idx_v)
    base = wid * per_w

    @pl.loop(0, n_chunks, step=NBUF)
    def group(i0):
        descs = []
        for b in range(NBUF):
            @pl.when(i0 > 0)
            def _(b=b):
                pltpu.make_async_copy(
                    rows_v.at[b], out_hbm.at[pl.ds(0, CH)], osems[b]
                ).wait()
            descs.append(
                pltpu.async_copy(
                    table_hbm.at[idx_v.at[i0 + b]], rows_v.at[b], gsems[b]
                )
            )
        for b in range(NBUF):
            descs[b].wait()
            row0 = base + (i0 + b) * CH
            pltpu.async_copy(
                rows_v.at[b], out_hbm.at[pl.ds(row0, CH)], osems[b]
            )

    for b in range(NBUF):
        pltpu.make_async_copy(
            rows_v.at[b], out_hbm.at[pl.ds(0, CH)], osems[b]
        ).wait()


def kernel(x, table):
    B, H = x.shape
    V, D = table.shape
    N = B * H
    per_w = N // NW
    n_chunks = per_w // CH
    assert per_w * NW == N and n_chunks * CH == per_w and n_chunks % NBUF == 0

    idx = x.reshape(NW, n_chunks, CH).astype(jnp.int32)
    mesh = plsc.VectorSubcoreMesh(core_axis_name="c", subcore_axis_name="s")

    run = pl.kernel(
        functools.partial(_gather_kernel, n_chunks, per_w, D),
        out_type=jax.ShapeDtypeStruct((N, D), jnp.float32),
        mesh=mesh,
        scratch_types=[
            pltpu.VMEM((n_chunks, CH), jnp.int32),
            pltpu.VMEM((NBUF, CH, D), jnp.float32),
            [pltpu.SemaphoreType.DMA] * NBUF,
            [pltpu.SemaphoreType.DMA] * NBUF,
        ],
        compiler_params=pltpu.CompilerParams(use_tc_tiling_on_sc=False),
    )
    out = run(idx, table)
    return out.reshape(B, H, D)
